# baseline (speedup 1.0000x reference)
import jax
import jax.numpy as jnp
from jax.experimental import pallas as pl
from jax.experimental.pallas import tpu as pltpu

_B, _MD, _CD, _H, _E = 4096, 138, 11, 512, 8
_ID = _MD + _CD
_BB = 512


def _elu(v):
    return jnp.where(v > 0, v, jnp.exp(jnp.minimum(v, 0.0)) - 1.0)


def _moe_body(x_ref, gW1_ref, gb1_ref, gW2_ref, gb2_ref,
              W0_ref, b0_ref, W1_ref, b1_ref, W2_ref, b2_ref,
              W3_ref, b3_ref, out_ref):
    f32 = jnp.float32
    bf = jnp.bfloat16
    x = x_ref[...]

    h = jnp.dot(x, gW1_ref[...], preferred_element_type=f32) + gb1_ref[...]
    h = _elu(h)
    logits = (jnp.dot(h.astype(bf), gW2_ref[...], preferred_element_type=f32)
              + gb2_ref[...])
    m = jnp.max(logits, axis=-1, keepdims=True)
    p = jnp.exp(logits - m)
    coeffs = p / jnp.sum(p, axis=-1, keepdims=True)
    cb = coeffs.astype(bf)

    def layer(inp_bf, W_ref, b_ref, act):
        acc = jnp.dot(cb, b_ref[...].astype(bf), preferred_element_type=f32)
        for e in range(_E):
            me = jnp.dot(inp_bf, W_ref[e], preferred_element_type=f32)
            acc = acc + coeffs[:, e:e + 1] * me
        if act:
            acc = _elu(acc)
        return acc

    y = layer(x, W0_ref, b0_ref, True)
    y = layer(y.astype(bf), W1_ref, b1_ref, True)
    y = layer(y.astype(bf), W2_ref, b2_ref, True)
    y = layer(y.astype(bf), W3_ref, b3_ref, False)
    out_ref[...] = y


def kernel(motion, command, gW1, gb1, gW2, gb2, W0, b0, W1, b1, W2, b2, W3, b3):
    bf = jnp.bfloat16
    x = jnp.concatenate([motion, command], axis=-1).astype(bf)
    gW1b = gW1.astype(bf)
    gW2b = gW2.astype(bf)
    Wt0 = W0.transpose(0, 2, 1).astype(bf)
    Wt1 = W1.transpose(0, 2, 1).astype(bf)
    Wt2 = W2.transpose(0, 2, 1).astype(bf)
    Wt3 = W3.transpose(0, 2, 1).astype(bf)

    grid = (_B // _BB,)
    const2 = lambda i: (0, 0)
    const3 = lambda i: (0, 0, 0)
    in_specs = [
        pl.BlockSpec((_BB, _ID), lambda i: (i, 0)),
        pl.BlockSpec((_ID, _H), const2),
        pl.BlockSpec((1, _H), const2),
        pl.BlockSpec((_H, _E), const2),
        pl.BlockSpec((1, _E), const2),
        pl.BlockSpec((_E, _ID, _H), const3),
        pl.BlockSpec((_E, _H), const2),
        pl.BlockSpec((_E, _H, _H), const3),
        pl.BlockSpec((_E, _H), const2),
        pl.BlockSpec((_E, _H, _H), const3),
        pl.BlockSpec((_E, _H), const2),
        pl.BlockSpec((_E, _H, _MD), const3),
        pl.BlockSpec((_E, _MD), const2),
    ]
    out = pl.pallas_call(
        _moe_body,
        grid=grid,
        in_specs=in_specs,
        out_specs=pl.BlockSpec((_BB, _MD), lambda i: (i, 0)),
        out_shape=jax.ShapeDtypeStruct((_B, _MD), jnp.float32),
        compiler_params=pltpu.CompilerParams(
            dimension_semantics=("parallel",),
        ),
    )(x, gW1b, gb1.reshape(1, _H), gW2b, gb2.reshape(1, _E),
      Wt0, b0, Wt1, b1, Wt2, b2, Wt3, b3)
    return out
